# Initial kernel scaffold; baseline (speedup 1.0000x reference)
import jax
import jax.numpy as jnp
from jax.experimental import pallas as pl


def kernel(x, Wg, bg, We, be):
    raise NotImplementedError("write your pallas kernel here")


# speedup vs baseline: 1.2283x; 1.2283x over previous
import functools

import jax
import jax.numpy as jnp
from jax.experimental import pallas as pl
from jax.experimental.pallas import tpu as pltpu

TAU = 0.8
LAM = 0.05
E = 8
D = 2048
B = 4096

BB = 2048
BH = 256
NB = B // BB
NH = D // BH


def _moe_body(x_ref, wg_ref, bg_ref, we_ref, be_ref, out_ref, aux_ref,
              acc_ref, probs_ref, psum_ref):
    i = pl.program_id(0)
    j = pl.program_id(1)
    e = pl.program_id(2)

    @pl.when(jnp.logical_and(j == 0, e == 0))
    def _gating():
        logits = (jnp.dot(x_ref[...].astype(jnp.float32), wg_ref[...],
                          preferred_element_type=jnp.float32)
                  + bg_ref[...]) / TAU
        m = jnp.max(logits, axis=1, keepdims=True)
        ex = jnp.exp(logits - m)
        p = ex / jnp.sum(ex, axis=1, keepdims=True)
        probs_ref[...] = p

        @pl.when(i == 0)
        def _():
            psum_ref[...] = jnp.zeros_like(psum_ref)

        psum_ref[...] += jnp.sum(p, axis=0, keepdims=True)

    @pl.when(e == 0)
    def _bias():
        acc_ref[...] = jnp.dot(probs_ref[...], be_ref[...],
                               preferred_element_type=jnp.float32)

    mask = (jax.lax.broadcasted_iota(jnp.int32, (1, E), 1) == e
            ).astype(jnp.float32)
    col = jnp.sum(probs_ref[...] * mask, axis=1, keepdims=True)
    acc_ref[...] += col * jnp.dot(x_ref[...], we_ref[0],
                                  preferred_element_type=jnp.float32)

    @pl.when(e == E - 1)
    def _flush():
        out_ref[...] = acc_ref[...]

    last = jnp.logical_and(i == NB - 1,
                           jnp.logical_and(j == NH - 1, e == E - 1))

    @pl.when(last)
    def _aux():
        mvec = psum_ref[...] / B
        mean_m = jnp.sum(mvec) / E
        var = jnp.sum((mvec - mean_m) ** 2) / (E - 1)
        cv = jnp.sqrt(var) / (mean_m + 1e-8)
        switch = E * jnp.sum(mvec * mvec)
        aux_ref[...] = jnp.full((1, 1), (switch + 2.0 * cv) * LAM,
                                dtype=jnp.float32)


def kernel(x, Wg, bg, We, be):
    bg2 = bg.reshape(1, E)
    out, aux = pl.pallas_call(
        _moe_body,
        grid=(NB, NH, E),
        in_specs=[
            pl.BlockSpec((BB, D), lambda i, j, e: (i, 0)),
            pl.BlockSpec((D, E), lambda i, j, e: (0, 0)),
            pl.BlockSpec((1, E), lambda i, j, e: (0, 0)),
            pl.BlockSpec((1, D, BH), lambda i, j, e: (e, 0, j)),
            pl.BlockSpec((E, BH), lambda i, j, e: (0, j)),
        ],
        out_specs=[
            pl.BlockSpec((BB, BH), lambda i, j, e: (i, j)),
            pl.BlockSpec((1, 1), lambda i, j, e: (0, 0)),
        ],
        out_shape=[
            jax.ShapeDtypeStruct((B, D), jnp.float32),
            jax.ShapeDtypeStruct((1, 1), jnp.float32),
        ],
        scratch_shapes=[
            pltpu.VMEM((BB, BH), jnp.float32),
            pltpu.VMEM((BB, E), jnp.float32),
            pltpu.VMEM((1, E), jnp.float32),
        ],
        compiler_params=pltpu.CompilerParams(
            dimension_semantics=("arbitrary", "arbitrary", "arbitrary"),
        ),
    )(x, Wg, bg2, We, be)
    return out, aux.reshape(())


# speedup vs baseline: 1.2582x; 1.0244x over previous
import jax
import jax.numpy as jnp
from jax.experimental import pallas as pl
from jax.experimental.pallas import tpu as pltpu

TAU = 0.8
LAM = 0.05
E = 8
D = 2048
B = 4096

BH = 256
NH = D // BH


def _moe_body(x_ref, wg_ref, bg_ref, we_ref, be_ref, out_ref, aux_ref,
              acc_ref, probs_ref):
    j = pl.program_id(0)
    e = pl.program_id(1)

    @pl.when(jnp.logical_and(j == 0, e == 0))
    def _gating():
        logits = (jnp.dot(x_ref[...], wg_ref[...],
                          preferred_element_type=jnp.float32)
                  + bg_ref[...]) / TAU
        m = jnp.max(logits, axis=1, keepdims=True)
        ex = jnp.exp(logits - m)
        p = ex / jnp.sum(ex, axis=1, keepdims=True)
        probs_ref[...] = p
        mvec = jnp.sum(p, axis=0, keepdims=True) / B
        mean_m = jnp.sum(mvec) / E
        var = jnp.sum((mvec - mean_m) ** 2) / (E - 1)
        cv = jnp.sqrt(var) / (mean_m + 1e-8)
        switch = E * jnp.sum(mvec * mvec)
        aux_ref[...] = jnp.full((1, 1), (switch + 2.0 * cv) * LAM,
                                dtype=jnp.float32)

    @pl.when(e == 0)
    def _bias():
        acc_ref[...] = jnp.dot(probs_ref[...], be_ref[...],
                               preferred_element_type=jnp.float32)

    mask = (jax.lax.broadcasted_iota(jnp.int32, (1, E), 1) == e
            ).astype(jnp.float32)
    col = jnp.sum(probs_ref[...] * mask, axis=1, keepdims=True)
    web = we_ref[0].astype(jnp.bfloat16)
    acc_ref[...] += col * jnp.dot(x_ref[...], web,
                                  preferred_element_type=jnp.float32)

    @pl.when(e == E - 1)
    def _flush():
        out_ref[...] = acc_ref[...]


def kernel(x, Wg, bg, We, be):
    xb = x.astype(jnp.bfloat16)
    bg2 = bg.reshape(1, E)
    out, aux = pl.pallas_call(
        _moe_body,
        grid=(NH, E),
        in_specs=[
            pl.BlockSpec((B, D), lambda j, e: (0, 0)),
            pl.BlockSpec((D, E), lambda j, e: (0, 0)),
            pl.BlockSpec((1, E), lambda j, e: (0, 0)),
            pl.BlockSpec((1, D, BH), lambda j, e: (e, 0, j)),
            pl.BlockSpec((E, BH), lambda j, e: (0, j)),
        ],
        out_specs=[
            pl.BlockSpec((B, BH), lambda j, e: (0, j)),
            pl.BlockSpec((1, 1), lambda j, e: (0, 0)),
        ],
        out_shape=[
            jax.ShapeDtypeStruct((B, D), jnp.float32),
            jax.ShapeDtypeStruct((1, 1), jnp.float32),
        ],
        scratch_shapes=[
            pltpu.VMEM((B, BH), jnp.float32),
            pltpu.VMEM((B, E), jnp.float32),
        ],
        compiler_params=pltpu.CompilerParams(
            dimension_semantics=("arbitrary", "arbitrary"),
        ),
    )(xb, Wg, bg2, We, be)
    return out, aux.reshape(())


# speedup vs baseline: 1.2740x; 1.0125x over previous
import jax
import jax.numpy as jnp
from jax.experimental import pallas as pl
from jax.experimental.pallas import tpu as pltpu

TAU = 0.8
LAM = 0.05
E = 8
D = 2048
B = 4096

BG = 512
NG = B // BG
BH = 256
NH = D // BH


def _gate_body(x_ref, wg_ref, bg_ref, xb_ref, probs_ref, aux_ref, psum_ref):
    i = pl.program_id(0)
    xblk = x_ref[...]
    xb_ref[...] = xblk.astype(jnp.bfloat16)
    logits = (jnp.dot(xblk, wg_ref[...], preferred_element_type=jnp.float32)
              + bg_ref[...]) / TAU
    m = jnp.max(logits, axis=1, keepdims=True)
    ex = jnp.exp(logits - m)
    p = ex / jnp.sum(ex, axis=1, keepdims=True)
    probs_ref[...] = p

    @pl.when(i == 0)
    def _():
        psum_ref[...] = jnp.zeros_like(psum_ref)

    psum_ref[...] += jnp.sum(p, axis=0, keepdims=True)

    @pl.when(i == NG - 1)
    def _aux():
        mvec = psum_ref[...] / B
        mean_m = jnp.sum(mvec) / E
        var = jnp.sum((mvec - mean_m) ** 2) / (E - 1)
        cv = jnp.sqrt(var) / (mean_m + 1e-8)
        switch = E * jnp.sum(mvec * mvec)
        aux_ref[...] = jnp.full((1, 1), (switch + 2.0 * cv) * LAM,
                                dtype=jnp.float32)


def _moe_body(xb_ref, probs_ref, we_ref, be_ref, out_ref, acc_ref):
    e = pl.program_id(1)

    @pl.when(e == 0)
    def _bias():
        acc_ref[...] = jnp.dot(probs_ref[...], be_ref[...],
                               preferred_element_type=jnp.float32)

    mask = (jax.lax.broadcasted_iota(jnp.int32, (1, E), 1) == e
            ).astype(jnp.float32)
    col = jnp.sum(probs_ref[...] * mask, axis=1, keepdims=True)
    web = we_ref[0].astype(jnp.bfloat16)
    acc_ref[...] += col * jnp.dot(xb_ref[...], web,
                                  preferred_element_type=jnp.float32)

    @pl.when(e == E - 1)
    def _flush():
        out_ref[...] = acc_ref[...]


def kernel(x, Wg, bg, We, be):
    bg2 = bg.reshape(1, E)
    xb, probs, aux = pl.pallas_call(
        _gate_body,
        grid=(NG,),
        in_specs=[
            pl.BlockSpec((BG, D), lambda i: (i, 0)),
            pl.BlockSpec((D, E), lambda i: (0, 0)),
            pl.BlockSpec((1, E), lambda i: (0, 0)),
        ],
        out_specs=[
            pl.BlockSpec((BG, D), lambda i: (i, 0)),
            pl.BlockSpec((BG, E), lambda i: (i, 0)),
            pl.BlockSpec((1, 1), lambda i: (0, 0)),
        ],
        out_shape=[
            jax.ShapeDtypeStruct((B, D), jnp.bfloat16),
            jax.ShapeDtypeStruct((B, E), jnp.float32),
            jax.ShapeDtypeStruct((1, 1), jnp.float32),
        ],
        scratch_shapes=[
            pltpu.VMEM((1, E), jnp.float32),
        ],
        compiler_params=pltpu.CompilerParams(
            dimension_semantics=("arbitrary",),
        ),
    )(x, Wg, bg2)

    out = pl.pallas_call(
        _moe_body,
        grid=(NH, E),
        in_specs=[
            pl.BlockSpec((B, D), lambda j, e: (0, 0)),
            pl.BlockSpec((B, E), lambda j, e: (0, 0)),
            pl.BlockSpec((1, D, BH), lambda j, e: (e, 0, j)),
            pl.BlockSpec((E, BH), lambda j, e: (0, j)),
        ],
        out_specs=pl.BlockSpec((B, BH), lambda j, e: (0, j)),
        out_shape=jax.ShapeDtypeStruct((B, D), jnp.float32),
        scratch_shapes=[
            pltpu.VMEM((B, BH), jnp.float32),
        ],
        compiler_params=pltpu.CompilerParams(
            dimension_semantics=("parallel", "arbitrary"),
        ),
    )(xb, probs, We, be)
    return out, aux.reshape(())


# speedup vs baseline: 1.2761x; 1.0017x over previous
import jax
import jax.numpy as jnp
from jax.experimental import pallas as pl
from jax.experimental.pallas import tpu as pltpu

TAU = 0.8
LAM = 0.05
E = 8
D = 2048
B = 4096

BG = 512
NG = B // BG
BH = 256
NH = D // BH


def _gate_body(x_ref, wg_ref, bg_ref, xb_ref, probs_ref, aux_ref, psum_ref):
    i = pl.program_id(0)
    xblk = x_ref[...]
    xb_ref[...] = xblk.astype(jnp.bfloat16)
    logits = (jnp.dot(xblk, wg_ref[...], preferred_element_type=jnp.float32)
              + bg_ref[...]) / TAU
    m = jnp.max(logits, axis=1, keepdims=True)
    ex = jnp.exp(logits - m)
    p = ex / jnp.sum(ex, axis=1, keepdims=True)
    probs_ref[...] = p

    @pl.when(i == 0)
    def _():
        psum_ref[...] = jnp.zeros_like(psum_ref)

    psum_ref[...] += jnp.sum(p, axis=0, keepdims=True)

    @pl.when(i == NG - 1)
    def _aux():
        mvec = psum_ref[...] / B
        mean_m = jnp.sum(mvec) / E
        var = jnp.sum((mvec - mean_m) ** 2) / (E - 1)
        cv = jnp.sqrt(var) / (mean_m + 1e-8)
        switch = E * jnp.sum(mvec * mvec)
        aux_ref[...] = jnp.full((1, 1), (switch + 2.0 * cv) * LAM,
                                dtype=jnp.float32)


def _moe_body(xb_ref, probs_ref, we_ref, be_ref, out_ref, acc_ref):
    e = pl.program_id(1)

    @pl.when(e == 0)
    def _bias():
        acc_ref[...] = jnp.dot(probs_ref[...], be_ref[...],
                               preferred_element_type=jnp.float32)

    mask = (jax.lax.broadcasted_iota(jnp.int32, (1, E), 1) == e
            ).astype(jnp.float32)
    col = jnp.sum(probs_ref[...] * mask, axis=1, keepdims=True)
    acc_ref[...] += col * jax.lax.dot_general(
        xb_ref[...], we_ref[0], (((1,), (0,)), ((), ())),
        preferred_element_type=jnp.float32)

    @pl.when(e == E - 1)
    def _flush():
        out_ref[...] = acc_ref[...]


def kernel(x, Wg, bg, We, be):
    bg2 = bg.reshape(1, E)
    xb, probs, aux = pl.pallas_call(
        _gate_body,
        grid=(NG,),
        in_specs=[
            pl.BlockSpec((BG, D), lambda i: (i, 0)),
            pl.BlockSpec((D, E), lambda i: (0, 0)),
            pl.BlockSpec((1, E), lambda i: (0, 0)),
        ],
        out_specs=[
            pl.BlockSpec((BG, D), lambda i: (i, 0)),
            pl.BlockSpec((BG, E), lambda i: (i, 0)),
            pl.BlockSpec((1, 1), lambda i: (0, 0)),
        ],
        out_shape=[
            jax.ShapeDtypeStruct((B, D), jnp.bfloat16),
            jax.ShapeDtypeStruct((B, E), jnp.float32),
            jax.ShapeDtypeStruct((1, 1), jnp.float32),
        ],
        scratch_shapes=[
            pltpu.VMEM((1, E), jnp.float32),
        ],
        compiler_params=pltpu.CompilerParams(
            dimension_semantics=("arbitrary",),
        ),
    )(x, Wg, bg2)

    out = pl.pallas_call(
        _moe_body,
        grid=(NH, E),
        in_specs=[
            pl.BlockSpec((B, D), lambda j, e: (0, 0)),
            pl.BlockSpec((B, E), lambda j, e: (0, 0)),
            pl.BlockSpec((1, D, BH), lambda j, e: (e, 0, j)),
            pl.BlockSpec((E, BH), lambda j, e: (0, j)),
        ],
        out_specs=pl.BlockSpec((B, BH), lambda j, e: (0, j)),
        out_shape=jax.ShapeDtypeStruct((B, D), jnp.float32),
        scratch_shapes=[
            pltpu.VMEM((B, BH), jnp.float32),
        ],
        compiler_params=pltpu.CompilerParams(
            dimension_semantics=("parallel", "arbitrary"),
        ),
    )(xb, probs, We, be)
    return out, aux.reshape(())


# speedup vs baseline: 1.2877x; 1.0091x over previous
import jax
import jax.numpy as jnp
from jax.experimental import pallas as pl
from jax.experimental.pallas import tpu as pltpu

TAU = 0.8
LAM = 0.05
E = 8
D = 2048
B = 4096

BG = 512
NG = B // BG
BH = 256
NH = D // BH


def _gate_body(x_ref, wg_ref, bg_ref, xb_ref, probs_ref, aux_ref, psum_ref):
    i = pl.program_id(0)
    xblk = x_ref[...]
    xb_ref[...] = xblk.astype(jnp.bfloat16)
    logits = (jnp.dot(xblk, wg_ref[...], preferred_element_type=jnp.float32)
              + bg_ref[...]) / TAU
    m = jnp.max(logits, axis=1, keepdims=True)
    ex = jnp.exp(logits - m)
    p = ex / jnp.sum(ex, axis=1, keepdims=True)
    probs_ref[...] = p

    @pl.when(i == 0)
    def _():
        psum_ref[...] = jnp.zeros_like(psum_ref)

    psum_ref[...] += jnp.sum(p, axis=0, keepdims=True)

    @pl.when(i == NG - 1)
    def _aux():
        mvec = psum_ref[...] / B
        mean_m = jnp.sum(mvec) / E
        var = jnp.sum((mvec - mean_m) ** 2) / (E - 1)
        cv = jnp.sqrt(var) / (mean_m + 1e-8)
        switch = E * jnp.sum(mvec * mvec)
        aux_ref[...] = jnp.full((1, 1), (switch + 2.0 * cv) * LAM,
                                dtype=jnp.float32)


def _moe_body(xb_ref, probs_ref, we_ref, be_ref, out_ref):
    e = pl.program_id(1)

    @pl.when(e == 0)
    def _bias():
        out_ref[...] = jnp.dot(probs_ref[...], be_ref[...],
                               preferred_element_type=jnp.float32)

    mask = (jax.lax.broadcasted_iota(jnp.int32, (1, E), 1) == e
            ).astype(jnp.float32)
    col = jnp.sum(probs_ref[...] * mask, axis=1, keepdims=True)
    out_ref[...] += col * jax.lax.dot_general(
        xb_ref[...], we_ref[0], (((1,), (0,)), ((), ())),
        preferred_element_type=jnp.float32)


def kernel(x, Wg, bg, We, be):
    bg2 = bg.reshape(1, E)
    xb, probs, aux = pl.pallas_call(
        _gate_body,
        grid=(NG,),
        in_specs=[
            pl.BlockSpec((BG, D), lambda i: (i, 0)),
            pl.BlockSpec((D, E), lambda i: (0, 0)),
            pl.BlockSpec((1, E), lambda i: (0, 0)),
        ],
        out_specs=[
            pl.BlockSpec((BG, D), lambda i: (i, 0)),
            pl.BlockSpec((BG, E), lambda i: (i, 0)),
            pl.BlockSpec((1, 1), lambda i: (0, 0)),
        ],
        out_shape=[
            jax.ShapeDtypeStruct((B, D), jnp.bfloat16),
            jax.ShapeDtypeStruct((B, E), jnp.float32),
            jax.ShapeDtypeStruct((1, 1), jnp.float32),
        ],
        scratch_shapes=[
            pltpu.VMEM((1, E), jnp.float32),
        ],
        compiler_params=pltpu.CompilerParams(
            dimension_semantics=("arbitrary",),
        ),
    )(x, Wg, bg2)

    out = pl.pallas_call(
        _moe_body,
        grid=(NH, E),
        in_specs=[
            pl.BlockSpec((B, D), lambda j, e: (0, 0)),
            pl.BlockSpec((B, E), lambda j, e: (0, 0)),
            pl.BlockSpec((1, D, BH), lambda j, e: (e, 0, j)),
            pl.BlockSpec((E, BH), lambda j, e: (0, j)),
        ],
        out_specs=pl.BlockSpec((B, BH), lambda j, e: (0, j)),
        out_shape=jax.ShapeDtypeStruct((B, D), jnp.float32),
        compiler_params=pltpu.CompilerParams(
            dimension_semantics=("parallel", "arbitrary"),
        ),
    )(xb, probs, We, be)
    return out, aux.reshape(())


# speedup vs baseline: 1.3530x; 1.0507x over previous
import jax
import jax.numpy as jnp
from jax.experimental import pallas as pl
from jax.experimental.pallas import tpu as pltpu

TAU = 0.8
LAM = 0.05
E = 8
D = 2048
B = 4096

BG = 512
NG = B // BG
BH = 512
NH = D // BH


def _gate_body(x_ref, wg_ref, bg_ref, xb_ref, probs_ref, aux_ref, psum_ref):
    i = pl.program_id(0)
    xblk = x_ref[...]
    xb_ref[...] = xblk.astype(jnp.bfloat16)
    logits = (jnp.dot(xblk, wg_ref[...], preferred_element_type=jnp.float32)
              + bg_ref[...]) / TAU
    m = jnp.max(logits, axis=1, keepdims=True)
    ex = jnp.exp(logits - m)
    p = ex / jnp.sum(ex, axis=1, keepdims=True)
    probs_ref[...] = p

    @pl.when(i == 0)
    def _():
        psum_ref[...] = jnp.zeros_like(psum_ref)

    psum_ref[...] += jnp.sum(p, axis=0, keepdims=True)

    @pl.when(i == NG - 1)
    def _aux():
        mvec = psum_ref[...] / B
        mean_m = jnp.sum(mvec) / E
        var = jnp.sum((mvec - mean_m) ** 2) / (E - 1)
        cv = jnp.sqrt(var) / (mean_m + 1e-8)
        switch = E * jnp.sum(mvec * mvec)
        aux_ref[...] = jnp.full((1, 1), (switch + 2.0 * cv) * LAM,
                                dtype=jnp.float32)


def _moe_body(xb_ref, probs_ref, we_ref, be_ref, out_ref):
    e = pl.program_id(1)

    @pl.when(e == 0)
    def _bias():
        out_ref[...] = jnp.dot(probs_ref[...], be_ref[...],
                               preferred_element_type=jnp.float32)

    mask = (jax.lax.broadcasted_iota(jnp.int32, (1, E), 1) == e
            ).astype(jnp.float32)
    col = jnp.sum(probs_ref[...] * mask, axis=1, keepdims=True)
    out_ref[...] += col * jax.lax.dot_general(
        xb_ref[...], we_ref[0], (((1,), (0,)), ((), ())),
        preferred_element_type=jnp.float32)


def kernel(x, Wg, bg, We, be):
    bg2 = bg.reshape(1, E)
    xb, probs, aux = pl.pallas_call(
        _gate_body,
        grid=(NG,),
        in_specs=[
            pl.BlockSpec((BG, D), lambda i: (i, 0)),
            pl.BlockSpec((D, E), lambda i: (0, 0)),
            pl.BlockSpec((1, E), lambda i: (0, 0)),
        ],
        out_specs=[
            pl.BlockSpec((BG, D), lambda i: (i, 0)),
            pl.BlockSpec((BG, E), lambda i: (i, 0)),
            pl.BlockSpec((1, 1), lambda i: (0, 0)),
        ],
        out_shape=[
            jax.ShapeDtypeStruct((B, D), jnp.bfloat16),
            jax.ShapeDtypeStruct((B, E), jnp.float32),
            jax.ShapeDtypeStruct((1, 1), jnp.float32),
        ],
        scratch_shapes=[
            pltpu.VMEM((1, E), jnp.float32),
        ],
        compiler_params=pltpu.CompilerParams(
            dimension_semantics=("arbitrary",),
        ),
    )(x, Wg, bg2)

    out = pl.pallas_call(
        _moe_body,
        grid=(NH, E),
        in_specs=[
            pl.BlockSpec((B, D), lambda j, e: (0, 0)),
            pl.BlockSpec((B, E), lambda j, e: (0, 0)),
            pl.BlockSpec((1, D, BH), lambda j, e: (e, 0, j)),
            pl.BlockSpec((E, BH), lambda j, e: (0, j)),
        ],
        out_specs=pl.BlockSpec((B, BH), lambda j, e: (0, j)),
        out_shape=jax.ShapeDtypeStruct((B, D), jnp.float32),
        compiler_params=pltpu.CompilerParams(
            dimension_semantics=("parallel", "arbitrary"),
        ),
    )(xb, probs, We, be)
    return out, aux.reshape(())
